# baseline (speedup 1.0000x reference)
import math

import jax
import jax.numpy as jnp
import numpy as np
from jax.experimental import pallas as pl
from jax.experimental.pallas import tpu as pltpu

_SIZE = 100000
_CONF = 0.9
_EPS = float(np.float32(0.1 / (_SIZE - 2)))
_H = _CONF * math.log(_CONF) + (_SIZE - 2) * _EPS * math.log(_EPS)
_ROWS_PER_BLOCK = 16
_STREAMS = 2


def _tc_body(t_ref, *refs):
    pid = pl.program_id(0)
    nsteps = pl.num_programs(0)
    x_refs = refs[:_STREAMS]
    o_ref = refs[_STREAMS]
    lane_iota = jax.lax.broadcasted_iota(jnp.int32, (1, 128), 1)
    contrib = 0.0
    for s in range(_STREAMS):
        x = x_refs[s][...]
        rowsum = jnp.sum(x, axis=1)
        row0 = (s * nsteps + pid) * _ROWS_PER_BLOCK
        for k in range(_ROWS_PER_BLOCK):
            tk = t_ref[row0 + k]
            wk = (tk != 0).astype(jnp.float32)
            col0 = pl.multiple_of((tk // 128) * 128, 128)
            window = x_refs[s][pl.ds(k, 1), pl.ds(col0, 128)]
            vk = jnp.sum(jnp.where(lane_iota == tk - col0, window, 0.0))
            x0k = x_refs[s][k, 0]
            contrib += wk * (
                _H - (_CONF - _EPS) * vk - _EPS * (rowsum[k] - x0k)
            )

    @pl.when(pid == 0)
    def _init():
        o_ref[0, 0] = 0.0

    o_ref[0, 0] += contrib


def kernel(x, target):
    n = x.shape[0]
    r = _ROWS_PER_BLOCK
    nsteps = n // (r * _STREAMS)
    t32 = target.astype(jnp.int32)
    x_specs = [
        pl.BlockSpec((r, _SIZE), lambda i, _s=s: (_s * nsteps + i, 0))
        for s in range(_STREAMS)
    ]
    out = pl.pallas_call(
        _tc_body,
        grid=(nsteps,),
        in_specs=[
            pl.BlockSpec((n,), lambda i: (0,), memory_space=pltpu.SMEM),
            *x_specs,
        ],
        out_specs=pl.BlockSpec(memory_space=pltpu.SMEM),
        out_shape=jax.ShapeDtypeStruct((1, 1), jnp.float32),
    )(t32, *([x] * _STREAMS))
    return out[0, 0]
